# baseline (speedup 1.0000x reference)
import functools

import jax
import jax.numpy as jnp
from jax import lax
from jax.experimental import pallas as pl
from jax.experimental.pallas import tpu as pltpu
from jax.experimental.pallas import tpu_sc as plsc

_EMBEDDING_DIM = 1024

_INFO = plsc.get_sparse_core_info()
_NC = _INFO.num_cores

_CH = 256
_NBUF = 3


def kernel(input, table):
    seq_len = input.shape[-1]
    rows_per_c = seq_len // _NC
    nchunks = rows_per_c // _CH
    mesh = plsc.ScalarSubcoreMesh(axis_name="c", num_cores=_NC)

    @functools.partial(
        pl.kernel,
        mesh=mesh,
        out_type=jax.ShapeDtypeStruct((1, seq_len, _EMBEDDING_DIM), table.dtype),
        scratch_types=[
            pltpu.VMEM_SHARED((_NBUF, _CH, _EMBEDDING_DIM), table.dtype),
            pltpu.SemaphoreType.DMA((_NBUF,)),
            pltpu.SemaphoreType.DMA((_NBUF,)),
        ],
    )
    def run(table_hbm, out_hbm, buf, in_sems, out_sems):
        cid = lax.axis_index("c")
        base = cid * rows_per_c

        def in_copy(c):
            return pltpu.make_async_copy(
                table_hbm.at[pl.ds(base + c * _CH, _CH)],
                buf.at[c % _NBUF],
                in_sems.at[c % _NBUF],
            )

        def out_copy(c):
            return pltpu.make_async_copy(
                buf.at[c % _NBUF],
                out_hbm.at[0].at[pl.ds(base + c * _CH, _CH)],
                out_sems.at[c % _NBUF],
            )

        for c in range(min(_NBUF, nchunks)):
            in_copy(c).start()
        for c in range(nchunks):
            in_copy(c).wait()
            out_copy(c).start()
            nxt = c + _NBUF
            if nxt < nchunks:
                out_copy(c).wait()
                in_copy(nxt).start()
        for c in range(max(nchunks - _NBUF, 0), nchunks):
            out_copy(c).wait()

    return run(table)


# speedup vs baseline: 1.0818x; 1.0818x over previous
import functools

import jax
import jax.numpy as jnp
from jax import lax
from jax.experimental import pallas as pl
from jax.experimental.pallas import tpu as pltpu
from jax.experimental.pallas import tpu_sc as plsc

_EMBEDDING_DIM = 1024

_INFO = plsc.get_sparse_core_info()
_NC, _NS = _INFO.num_cores, _INFO.num_subcores
_NW = _NC * _NS

_CH = 16
_NBUF_T = 3
_NBUF_S = 2


def kernel(input, table):
    seq_len = input.shape[-1]
    rows_per_w = seq_len // _NW
    half = rows_per_w // 2
    nchunks = half // _CH
    mesh = plsc.VectorSubcoreMesh(core_axis_name="c", subcore_axis_name="s")

    @functools.partial(
        pl.kernel,
        mesh=mesh,
        out_type=jax.ShapeDtypeStruct((1, seq_len, _EMBEDDING_DIM), table.dtype),
        scratch_types=[
            pltpu.VMEM((_NBUF_T, _CH, _EMBEDDING_DIM), table.dtype),
            pltpu.VMEM_SHARED((_NS, _NBUF_S, _CH, _EMBEDDING_DIM), table.dtype),
            pltpu.SemaphoreType.DMA((_NBUF_T,)),
            pltpu.SemaphoreType.DMA((_NBUF_T,)),
            pltpu.SemaphoreType.DMA((_NBUF_S,)),
            pltpu.SemaphoreType.DMA((_NBUF_S,)),
        ],
    )
    def run(table_hbm, out_hbm, buf_t, buf_s, in_t, out_t, in_s, out_s):
        sid = lax.axis_index("s")
        wid = sid * _NC + lax.axis_index("c")
        base_a = wid * rows_per_w
        base_b = base_a + half

        def copies(c, base, buf, isem, osem, nbuf):
            src = table_hbm.at[pl.ds(base + c * _CH, _CH)]
            dst = out_hbm.at[0].at[pl.ds(base + c * _CH, _CH)]
            stage = buf.at[c % nbuf]
            return (
                pltpu.make_async_copy(src, stage, isem.at[c % nbuf]),
                pltpu.make_async_copy(stage, dst, osem.at[c % nbuf]),
            )

        def path_a(c):
            return copies(c, base_a, buf_t, in_t, out_t, _NBUF_T)

        def path_b(c):
            return copies(c, base_b, buf_s.at[sid], in_s, out_s, _NBUF_S)

        paths = ((path_a, _NBUF_T), (path_b, _NBUF_S))
        for path, nbuf in paths:
            for c in range(min(nbuf, nchunks)):
                path(c)[0].start()
        for c in range(nchunks):
            for path, nbuf in paths:
                icp, ocp = path(c)
                icp.wait()
                ocp.start()
                nxt = c + nbuf
                if nxt < nchunks:
                    ocp.wait()
                    path(nxt)[0].start()
        for path, nbuf in paths:
            for c in range(max(nchunks - nbuf, 0), nchunks):
                path(c)[1].wait()

    return run(table)
